# initial kernel scaffold (unmeasured)
import functools

import jax
import jax.numpy as jnp
from jax import lax
from jax.experimental import pallas as pl
from jax.experimental.pallas import tpu as pltpu

N_DEV = 4
B, SQ, H, D = 4, 32, 8, 128
SKV = 4096
KC = 2048
N_KC = SKV // KC
SCALE = D ** -0.5

ROWS_A = B * SQ * H
ROWS_L = ROWS_A // 128
ROWS = ROWS_A + ROWS_L


def _flash_partial_body(q_ref, k_ref, v_ref, a_out, l_out, acc, lacc):
    kc = pl.program_id(1)

    @pl.when(kc == 0)
    def _():
        acc[...] = jnp.zeros_like(acc)
        lacc[...] = jnp.zeros_like(lacc)

    for h in range(H):
        q = (q_ref[0, :, h, :] * SCALE).astype(jnp.bfloat16)
        k = k_ref[0, :, h, :].astype(jnp.bfloat16)
        v = v_ref[0, :, h, :].astype(jnp.bfloat16)
        s = lax.dot_general(
            q, k, (((1,), (1,)), ((), ())),
            preferred_element_type=jnp.float32,
        )
        p = jnp.exp(s)
        lacc[:, h : h + 1] = lacc[:, h : h + 1] + jnp.sum(p, axis=1, keepdims=True)
        a = lax.dot_general(
            p.astype(jnp.bfloat16), v, (((1,), (0,)), ((), ())),
            preferred_element_type=jnp.float32,
        )
        acc[:, h, :] = acc[:, h, :] + a

    @pl.when(kc == N_KC - 1)
    def _():
        a_out[0] = acc[...].astype(jnp.bfloat16)
        l_out[0] = lacc[...].astype(jnp.bfloat16)


def _flash_partial(Q, K, V):
    return pl.pallas_call(
        _flash_partial_body,
        grid=(B, N_KC),
        in_specs=[
            pl.BlockSpec((1, SQ, H, D), lambda b, kc: (b, 0, 0, 0)),
            pl.BlockSpec((1, KC, H, D), lambda b, kc: (b, kc, 0, 0)),
            pl.BlockSpec((1, KC, H, D), lambda b, kc: (b, kc, 0, 0)),
        ],
        out_specs=[
            pl.BlockSpec((1, SQ, H, D), lambda b, kc: (b, 0, 0, 0)),
            pl.BlockSpec((1, SQ, H), lambda b, kc: (b, 0, 0)),
        ],
        out_shape=[
            jax.ShapeDtypeStruct((B, SQ, H, D), jnp.bfloat16),
            jax.ShapeDtypeStruct((B, SQ, H), jnp.bfloat16),
        ],
        scratch_shapes=[
            pltpu.VMEM((SQ, H, D), jnp.float32),
            pltpu.VMEM((SQ, H), jnp.float32),
        ],
        compiler_params=pltpu.CompilerParams(
            dimension_semantics=("arbitrary", "arbitrary"),
        ),
    )(Q, K, V)


def _allreduce_body(a_ref, l_ref, oa_ref, ol_ref, sbuf, rbuf, send_sems, recv_sems):
    my = lax.axis_index("i")
    p1 = my ^ 1
    p2 = 3 - my

    barrier_sem = pltpu.get_barrier_semaphore()
    for nbr in (p1, p2):
        pl.semaphore_signal(
            barrier_sem, inc=1,
            device_id=(nbr,), device_id_type=pl.DeviceIdType.MESH,
        )
    pl.semaphore_wait(barrier_sem, 2)

    sbuf[0:ROWS_A, :] = a_ref[...]
    sbuf[ROWS_A:ROWS, :] = l_ref[...]

    rdma0 = pltpu.make_async_remote_copy(
        src_ref=sbuf,
        dst_ref=rbuf.at[0],
        send_sem=send_sems.at[0],
        recv_sem=recv_sems.at[0],
        device_id=(p1,),
        device_id_type=pl.DeviceIdType.MESH,
    )
    rdma0.start()
    rdma0.wait()
    sbuf[...] = sbuf[...] + rbuf[0]

    rdma1 = pltpu.make_async_remote_copy(
        src_ref=sbuf,
        dst_ref=rbuf.at[1],
        send_sem=send_sems.at[1],
        recv_sem=recv_sems.at[1],
        device_id=(p2,),
        device_id_type=pl.DeviceIdType.MESH,
    )
    rdma1.start()
    rdma1.wait()
    tot = sbuf[...] + rbuf[1]
    oa_ref[...] = tot[0:ROWS_A, :]
    ol_ref[...] = tot[ROWS_A:ROWS, :]


def _allreduce(A2, l2):
    return pl.pallas_call(
        _allreduce_body,
        in_specs=[
            pl.BlockSpec(memory_space=pltpu.VMEM),
            pl.BlockSpec(memory_space=pltpu.VMEM),
        ],
        out_specs=[
            pl.BlockSpec(memory_space=pltpu.VMEM),
            pl.BlockSpec(memory_space=pltpu.VMEM),
        ],
        out_shape=[
            jax.ShapeDtypeStruct((ROWS_A, 128), jnp.bfloat16),
            jax.ShapeDtypeStruct((ROWS_L, 128), jnp.bfloat16),
        ],
        scratch_shapes=[
            pltpu.VMEM((ROWS, 128), jnp.bfloat16),
            pltpu.VMEM((2, ROWS, 128), jnp.bfloat16),
            pltpu.SemaphoreType.DMA((2,)),
            pltpu.SemaphoreType.DMA((2,)),
        ],
        compiler_params=pltpu.CompilerParams(collective_id=0),
    )(A2, l2)


def kernel(Q, K, V):
    A, l = _flash_partial(Q, K, V)
    As, ls = _allreduce(A.reshape(ROWS_A, 128), l.reshape(ROWS_L, 128))
    O = As.astype(jnp.float32).reshape(B, SQ, H, D)
    den = ls.astype(jnp.float32).reshape(B, SQ, H)[..., None]
    return O / den


# baseline (device time: 158933 ns/iter reference)
import functools

import jax
import jax.numpy as jnp
from jax import lax
from jax.experimental import pallas as pl
from jax.experimental.pallas import tpu as pltpu

N_DEV = 4
B, SQ, H, D = 4, 32, 8, 128
SKV = 4096
KC = 1024
N_KC = SKV // KC
SCALE = D ** -0.5

ROWS_A = B * SQ * H
ROWS_L = ROWS_A // 128
ROWS = ROWS_A + ROWS_L


def _flash_partial_body(q_ref, k_ref, v_ref, a_out, l_out, acc, lacc):
    kc = pl.program_id(1)

    @pl.when(kc == 0)
    def _():
        acc[...] = jnp.zeros_like(acc)
        lacc[...] = jnp.zeros_like(lacc)

    for h in range(H):
        q = (q_ref[0, :, h, :] * SCALE).astype(jnp.bfloat16)
        k = k_ref[0, :, h, :].astype(jnp.bfloat16)
        v = v_ref[0, :, h, :].astype(jnp.bfloat16)
        s = lax.dot_general(
            q, k, (((1,), (1,)), ((), ())),
            preferred_element_type=jnp.float32,
        )
        p = jnp.exp(s)
        lacc[:, h : h + 1] = lacc[:, h : h + 1] + jnp.sum(p, axis=1, keepdims=True)
        a = lax.dot_general(
            p.astype(jnp.bfloat16), v, (((1,), (0,)), ((), ())),
            preferred_element_type=jnp.float32,
        )
        acc[:, h, :] = acc[:, h, :] + a

    @pl.when(kc == N_KC - 1)
    def _():
        a_out[0] = acc[...].astype(jnp.bfloat16)
        l_out[0] = lacc[...].astype(jnp.bfloat16)


def _flash_partial(Q, K, V):
    return pl.pallas_call(
        _flash_partial_body,
        grid=(B, N_KC),
        in_specs=[
            pl.BlockSpec((1, SQ, H, D), lambda b, kc: (b, 0, 0, 0)),
            pl.BlockSpec((1, KC, H, D), lambda b, kc: (b, kc, 0, 0)),
            pl.BlockSpec((1, KC, H, D), lambda b, kc: (b, kc, 0, 0)),
        ],
        out_specs=[
            pl.BlockSpec((1, SQ, H, D), lambda b, kc: (b, 0, 0, 0)),
            pl.BlockSpec((1, SQ, H), lambda b, kc: (b, 0, 0)),
        ],
        out_shape=[
            jax.ShapeDtypeStruct((B, SQ, H, D), jnp.bfloat16),
            jax.ShapeDtypeStruct((B, SQ, H), jnp.bfloat16),
        ],
        scratch_shapes=[
            pltpu.VMEM((SQ, H, D), jnp.float32),
            pltpu.VMEM((SQ, H), jnp.float32),
        ],
        compiler_params=pltpu.CompilerParams(
            dimension_semantics=("arbitrary", "arbitrary"),
        ),
    )(Q, K, V)


def _allreduce_body(a_ref, l_ref, oa_ref, ol_ref, sbuf, rbuf, send_sems, recv_sems):
    my = lax.axis_index("i")
    p1 = my ^ 1
    p2 = 3 - my

    barrier_sem = pltpu.get_barrier_semaphore()
    for nbr in (p1, p2):
        pl.semaphore_signal(
            barrier_sem, inc=1,
            device_id=(nbr,), device_id_type=pl.DeviceIdType.MESH,
        )
    pl.semaphore_wait(barrier_sem, 2)

    sbuf[0:ROWS_A, :] = a_ref[...]
    sbuf[ROWS_A:ROWS, :] = l_ref[...]

    rdma0 = pltpu.make_async_remote_copy(
        src_ref=sbuf,
        dst_ref=rbuf.at[0],
        send_sem=send_sems.at[0],
        recv_sem=recv_sems.at[0],
        device_id=(p1,),
        device_id_type=pl.DeviceIdType.MESH,
    )
    rdma0.start()
    rdma0.wait()
    sbuf[...] = sbuf[...] + rbuf[0]

    rdma1 = pltpu.make_async_remote_copy(
        src_ref=sbuf,
        dst_ref=rbuf.at[1],
        send_sem=send_sems.at[1],
        recv_sem=recv_sems.at[1],
        device_id=(p2,),
        device_id_type=pl.DeviceIdType.MESH,
    )
    rdma1.start()
    rdma1.wait()
    tot = sbuf[...] + rbuf[1]
    oa_ref[...] = tot[0:ROWS_A, :]
    ol_ref[...] = tot[ROWS_A:ROWS, :]


def _allreduce(A2, l2):
    return pl.pallas_call(
        _allreduce_body,
        in_specs=[
            pl.BlockSpec(memory_space=pltpu.VMEM),
            pl.BlockSpec(memory_space=pltpu.VMEM),
        ],
        out_specs=[
            pl.BlockSpec(memory_space=pltpu.VMEM),
            pl.BlockSpec(memory_space=pltpu.VMEM),
        ],
        out_shape=[
            jax.ShapeDtypeStruct((ROWS_A, 128), jnp.bfloat16),
            jax.ShapeDtypeStruct((ROWS_L, 128), jnp.bfloat16),
        ],
        scratch_shapes=[
            pltpu.VMEM((ROWS, 128), jnp.bfloat16),
            pltpu.VMEM((2, ROWS, 128), jnp.bfloat16),
            pltpu.SemaphoreType.DMA((2,)),
            pltpu.SemaphoreType.DMA((2,)),
        ],
        compiler_params=pltpu.CompilerParams(collective_id=0),
    )(A2, l2)


def kernel(Q, K, V):
    A, l = _flash_partial(Q, K, V)
    As, ls = _allreduce(A.reshape(ROWS_A, 128), l.reshape(ROWS_L, 128))
    O = As.astype(jnp.float32).reshape(B, SQ, H, D)
    den = ls.astype(jnp.float32).reshape(B, SQ, H)[..., None]
    return O / den


# device time: 63891 ns/iter; 2.4876x vs baseline; 2.4876x over previous
import functools

import jax
import jax.numpy as jnp
from jax import lax
from jax.experimental import pallas as pl
from jax.experimental.pallas import tpu as pltpu

N_DEV = 4
B, SQ, H, D = 4, 32, 8, 128
SKV = 4096
SCALE = D ** -0.5

ROWS_A = B * SQ * H
ROWS_L = ROWS_A // 128
ROWS = ROWS_A + ROWS_L


def _flash_partial_body(q_ref, k_hbm, v_hbm, a_out, l_out, kbuf, vbuf, ksem, vsem):
    b = pl.program_id(0)
    h = pl.program_id(1)
    n = b * H + h
    slot = lax.rem(n, 2)
    nslot = lax.rem(n + 1, 2)

    def k_copy(bb, hh, s):
        return pltpu.make_async_copy(k_hbm.at[bb, :, hh, :], kbuf.at[s], ksem.at[s])

    def v_copy(bb, hh, s):
        return pltpu.make_async_copy(v_hbm.at[bb, :, hh, :], vbuf.at[s], vsem.at[s])

    @pl.when(n == 0)
    def _():
        k_copy(b, h, slot).start()
        v_copy(b, h, slot).start()

    @pl.when(n + 1 < B * H)
    def _():
        nb = lax.div(n + 1, H)
        nh = lax.rem(n + 1, H)
        k_copy(nb, nh, nslot).start()
        v_copy(nb, nh, nslot).start()

    k_copy(b, h, slot).wait()
    v_copy(b, h, slot).wait()

    q = q_ref[0, 0]
    k = kbuf[slot]
    v = vbuf[slot]
    s = lax.dot_general(
        q, k, (((1,), (1,)), ((), ())),
        preferred_element_type=jnp.float32,
    )
    p = jnp.exp(s)
    l = jnp.sum(p, axis=1, keepdims=True)
    a = lax.dot_general(
        p, v, (((1,), (0,)), ((), ())),
        preferred_element_type=jnp.float32,
    )
    a_out[0, 0] = a.astype(jnp.bfloat16)
    l_out[0, 0] = l.astype(jnp.bfloat16)


def _flash_partial(Qt, K, V):
    return pl.pallas_call(
        _flash_partial_body,
        grid=(B, H),
        in_specs=[
            pl.BlockSpec((1, 1, SQ, D), lambda b, h: (b, h, 0, 0)),
            pl.BlockSpec(memory_space=pl.ANY),
            pl.BlockSpec(memory_space=pl.ANY),
        ],
        out_specs=[
            pl.BlockSpec((1, 1, SQ, D), lambda b, h: (b, h, 0, 0)),
            pl.BlockSpec((1, 1, SQ, 1), lambda b, h: (b, h, 0, 0)),
        ],
        out_shape=[
            jax.ShapeDtypeStruct((B, H, SQ, D), jnp.bfloat16),
            jax.ShapeDtypeStruct((B, H, SQ, 1), jnp.bfloat16),
        ],
        scratch_shapes=[
            pltpu.VMEM((2, SKV, D), jnp.float32),
            pltpu.VMEM((2, SKV, D), jnp.float32),
            pltpu.SemaphoreType.DMA((2,)),
            pltpu.SemaphoreType.DMA((2,)),
        ],
        compiler_params=pltpu.CompilerParams(
            dimension_semantics=("arbitrary", "arbitrary"),
        ),
    )(Qt, K, V)


def _allreduce_body(a_ref, l_ref, oa_ref, ol_ref, sbuf, rbuf, send_sems, recv_sems):
    my = lax.axis_index("i")
    p1 = my ^ 1
    p2 = 3 - my

    barrier_sem = pltpu.get_barrier_semaphore()
    for nbr in (p1, p2):
        pl.semaphore_signal(
            barrier_sem, inc=1,
            device_id=(nbr,), device_id_type=pl.DeviceIdType.MESH,
        )
    pl.semaphore_wait(barrier_sem, 2)

    sbuf[0:ROWS_A, :] = a_ref[...]
    sbuf[ROWS_A:ROWS, :] = l_ref[...]

    rdma0 = pltpu.make_async_remote_copy(
        src_ref=sbuf,
        dst_ref=rbuf.at[0],
        send_sem=send_sems.at[0],
        recv_sem=recv_sems.at[0],
        device_id=(p1,),
        device_id_type=pl.DeviceIdType.MESH,
    )
    rdma0.start()
    rdma0.wait()
    sbuf[...] = sbuf[...] + rbuf[0]

    rdma1 = pltpu.make_async_remote_copy(
        src_ref=sbuf,
        dst_ref=rbuf.at[1],
        send_sem=send_sems.at[1],
        recv_sem=recv_sems.at[1],
        device_id=(p2,),
        device_id_type=pl.DeviceIdType.MESH,
    )
    rdma1.start()
    rdma1.wait()
    tot = sbuf[...] + rbuf[1]
    oa_ref[...] = tot[0:ROWS_A, :]
    ol_ref[...] = tot[ROWS_A:ROWS, :]


def _allreduce(A2, l2):
    return pl.pallas_call(
        _allreduce_body,
        in_specs=[
            pl.BlockSpec(memory_space=pltpu.VMEM),
            pl.BlockSpec(memory_space=pltpu.VMEM),
        ],
        out_specs=[
            pl.BlockSpec(memory_space=pltpu.VMEM),
            pl.BlockSpec(memory_space=pltpu.VMEM),
        ],
        out_shape=[
            jax.ShapeDtypeStruct((ROWS_A, 128), jnp.bfloat16),
            jax.ShapeDtypeStruct((ROWS_L, 128), jnp.bfloat16),
        ],
        scratch_shapes=[
            pltpu.VMEM((ROWS, 128), jnp.bfloat16),
            pltpu.VMEM((2, ROWS, 128), jnp.bfloat16),
            pltpu.SemaphoreType.DMA((2,)),
            pltpu.SemaphoreType.DMA((2,)),
        ],
        compiler_params=pltpu.CompilerParams(collective_id=0),
    )(A2, l2)


def kernel(Q, K, V):
    Qt = Q.transpose(0, 2, 1, 3) * SCALE
    A, l = _flash_partial(Qt, K, V)
    As, ls = _allreduce(A.reshape(ROWS_A, 128), l.reshape(ROWS_L, 128))
    O = As.astype(jnp.float32).reshape(B, H, SQ, D)
    den = ls.astype(jnp.float32).reshape(B, H, SQ)[..., None]
    return (O / den).transpose(0, 2, 1, 3)
